# baseline (speedup 1.0000x reference)
import functools

import jax
import jax.numpy as jnp
from jax import lax
from jax.experimental import pallas as pl
from jax.experimental.pallas import tpu as pltpu
from jax.experimental.pallas import tpu_sc as plsc

N_TYPES = 100
D = 512
B = 100000
NC = 2
NS = 16
NW = NC * NS
C = 64
NSLOT = 50
RPW = NSLOT * C
LAST_N = B - (NW - 1) * RPW
TAIL = LAST_N - (LAST_N // C) * C
L = 16


def _emb_body(idx_hbm, w_hbm, out_hbm, table_v, idx_v, rows0, rows1, tsem, o0, o1):
    wid = lax.axis_index("s") * NC + lax.axis_index("c")
    base = wid * RPW
    nval = jnp.where(wid == NW - 1, LAST_N // C, NSLOT)

    pltpu.async_copy(w_hbm, table_v, tsem)

    @pl.when(wid == NW - 1)
    def _():
        pltpu.sync_copy(idx_hbm.at[pl.ds(base, LAST_N)], idx_v.at[pl.ds(0, LAST_N)])

    @pl.when(wid != NW - 1)
    def _():
        pltpu.sync_copy(idx_hbm.at[pl.ds(base, RPW)], idx_v)

    pltpu.make_async_copy(w_hbm, table_v, tsem).wait()

    def expand16(idx16, rows, dst_base):
        for lane in range(L):
            src = idx16[lane] * D
            dst = dst_base + lane * D
            for c in range(0, D, L):
                rows[pl.ds(dst + c, L)] = table_v[pl.ds(src + c, L)]

    def compute(j, rows):
        @plsc.parallel_loop(0, C, unroll=8)
        def _(r):
            t = idx_v[pl.ds(j * C + r, L)][0]
            src = t * D
            dst = r * D
            for c in range(0, D, L):
                rows[pl.ds(dst + c, L)] = table_v[pl.ds(src + c, L)]

    def scatter_start(j, rows, sem):
        pltpu.async_copy(rows, out_hbm.at[pl.ds((base + j * C) * D, C * D)], sem)

    def scatter_wait(rows, sem):
        pltpu.make_async_copy(rows, out_hbm.at[pl.ds(base * D, C * D)], sem).wait()

    def step(t, carry):
        j0 = 2 * t
        j1 = j0 + 1

        @pl.when(t > 0)
        def _():
            scatter_wait(rows0, o0)

        compute(j0, rows0)
        scatter_start(j0, rows0, o0)

        @pl.when(t > 0)
        def _():
            scatter_wait(rows1, o1)

        compute(j1, rows1)
        scatter_start(j1, rows1, o1)
        return carry

    lax.fori_loop(0, nval // 2, step, 0)
    scatter_wait(rows0, o0)
    scatter_wait(rows1, o1)

    @pl.when(wid == NW - 1)
    def _():
        tbase = (LAST_N // C) * C
        for g in range(TAIL // L):
            idx16 = idx_v[pl.ds(tbase + g * L, L)]
            expand16(idx16, rows0, g * L * D)
        pltpu.async_copy(
            rows0.at[pl.ds(0, TAIL * D)],
            out_hbm.at[pl.ds((base + tbase) * D, TAIL * D)],
            o0,
        )
        pltpu.make_async_copy(
            rows0.at[pl.ds(0, TAIL * D)],
            out_hbm.at[pl.ds((base + tbase) * D, TAIL * D)],
            o0,
        ).wait()


@jax.jit
def _emb(idx, w):
    mesh = plsc.VectorSubcoreMesh(core_axis_name="c", subcore_axis_name="s")
    f = functools.partial(
        pl.kernel,
        mesh=mesh,
        out_type=jax.ShapeDtypeStruct((B * D,), jnp.float32),
        scratch_types=[
            pltpu.VMEM((N_TYPES * D,), jnp.float32),
            pltpu.VMEM((RPW,), jnp.int32),
            pltpu.VMEM((C * D,), jnp.float32),
            pltpu.VMEM((C * D,), jnp.float32),
            pltpu.SemaphoreType.DMA,
            pltpu.SemaphoreType.DMA,
            pltpu.SemaphoreType.DMA,
        ],
    )(_emb_body)
    return f(idx, w)


def kernel(atom_numbers, W):
    idx = jnp.squeeze(atom_numbers, axis=-1)
    out = _emb(idx, W.reshape(-1))
    return out.reshape(B, D)


# speedup vs baseline: 1.0031x; 1.0031x over previous
import functools

import jax
import jax.numpy as jnp
from jax import lax
from jax.experimental import pallas as pl
from jax.experimental.pallas import tpu as pltpu
from jax.experimental.pallas import tpu_sc as plsc

N_TYPES = 100
D = 512
B = 100000
NC = 2
NS = 16
NW = NC * NS
C = 64
NSLOT = 50
RPW = NSLOT * C
LAST_N = B - (NW - 1) * RPW
TAIL = LAST_N - (LAST_N // C) * C
L = 16


def _emb_body(idx_hbm, w_hbm, out_hbm, table_v, idx_v, rows0, rows1, tsem, o0, o1):
    wid = lax.axis_index("s") * NC + lax.axis_index("c")
    base = wid * RPW
    nval = jnp.where(wid == NW - 1, LAST_N // C, NSLOT)

    pltpu.async_copy(w_hbm, table_v, tsem)

    @pl.when(wid == NW - 1)
    def _():
        pltpu.sync_copy(idx_hbm.at[pl.ds(base, LAST_N)], idx_v.at[pl.ds(0, LAST_N)])

    @pl.when(wid != NW - 1)
    def _():
        pltpu.sync_copy(idx_hbm.at[pl.ds(base, RPW)], idx_v)

    pltpu.make_async_copy(w_hbm, table_v, tsem).wait()

    def expand16(idx16, rows, dst_base):
        for lane in range(L):
            src = idx16[lane] * D
            dst = dst_base + lane * D
            for cb in range(0, D, L * 8):
                vals = [table_v[pl.ds(src + cb + k * L, L)] for k in range(8)]
                for k in range(8):
                    rows[pl.ds(dst + cb + k * L, L)] = vals[k]

    def compute(j, rows):
        @plsc.parallel_loop(0, C, unroll=2)
        def _(r):
            t = idx_v[pl.ds(j * C + r, L)][0]
            src = t * D
            dst = r * D
            for cb in range(0, D, L * 8):
                vals = [table_v[pl.ds(src + cb + k * L, L)] for k in range(8)]
                for k in range(8):
                    rows[pl.ds(dst + cb + k * L, L)] = vals[k]

    def scatter_start(j, rows, sem):
        pltpu.async_copy(rows, out_hbm.at[pl.ds((base + j * C) * D, C * D)], sem)

    def scatter_wait(rows, sem):
        pltpu.make_async_copy(rows, out_hbm.at[pl.ds(base * D, C * D)], sem).wait()

    def step(t, carry):
        j0 = 2 * t
        j1 = j0 + 1

        @pl.when(t > 0)
        def _():
            scatter_wait(rows0, o0)

        compute(j0, rows0)
        scatter_start(j0, rows0, o0)

        @pl.when(t > 0)
        def _():
            scatter_wait(rows1, o1)

        compute(j1, rows1)
        scatter_start(j1, rows1, o1)
        return carry

    lax.fori_loop(0, nval // 2, step, 0)
    scatter_wait(rows0, o0)
    scatter_wait(rows1, o1)

    @pl.when(wid == NW - 1)
    def _():
        tbase = (LAST_N // C) * C
        for g in range(TAIL // L):
            idx16 = idx_v[pl.ds(tbase + g * L, L)]
            expand16(idx16, rows0, g * L * D)
        pltpu.async_copy(
            rows0.at[pl.ds(0, TAIL * D)],
            out_hbm.at[pl.ds((base + tbase) * D, TAIL * D)],
            o0,
        )
        pltpu.make_async_copy(
            rows0.at[pl.ds(0, TAIL * D)],
            out_hbm.at[pl.ds((base + tbase) * D, TAIL * D)],
            o0,
        ).wait()


@jax.jit
def _emb(idx, w):
    mesh = plsc.VectorSubcoreMesh(core_axis_name="c", subcore_axis_name="s")
    f = functools.partial(
        pl.kernel,
        mesh=mesh,
        out_type=jax.ShapeDtypeStruct((B * D,), jnp.float32),
        scratch_types=[
            pltpu.VMEM((N_TYPES * D,), jnp.float32),
            pltpu.VMEM((RPW,), jnp.int32),
            pltpu.VMEM((C * D,), jnp.float32),
            pltpu.VMEM((C * D,), jnp.float32),
            pltpu.SemaphoreType.DMA,
            pltpu.SemaphoreType.DMA,
            pltpu.SemaphoreType.DMA,
        ],
    )(_emb_body)
    return f(idx, w)


def kernel(atom_numbers, W):
    idx = jnp.squeeze(atom_numbers, axis=-1)
    out = _emb(idx, W.reshape(-1))
    return out.reshape(B, D)


# speedup vs baseline: 1.0074x; 1.0043x over previous
import functools

import jax
import jax.numpy as jnp
from jax import lax
from jax.experimental import pallas as pl
from jax.experimental.pallas import tpu as pltpu
from jax.experimental.pallas import tpu_sc as plsc

N_TYPES = 100
D = 512
B = 100000
NC = 2
NS = 16
NW = NC * NS
C = 64
NSLOT = 50
RPW = NSLOT * C
LAST_N = B - (NW - 1) * RPW
TAIL = LAST_N - (LAST_N // C) * C
L = 16
DP = D + L


def _emb_body(idx_hbm, w_hbm, out_hbm, table_v, idx_v, rows0, rows1, tsem, o0, o1):
    wid = lax.axis_index("s") * NC + lax.axis_index("c")
    base = wid * RPW
    nval = jnp.where(wid == NW - 1, LAST_N // C, NSLOT)

    pltpu.async_copy(w_hbm, table_v, tsem)

    @pl.when(wid == NW - 1)
    def _():
        pltpu.sync_copy(idx_hbm.at[pl.ds(base, LAST_N)], idx_v.at[pl.ds(0, LAST_N)])

    @pl.when(wid != NW - 1)
    def _():
        pltpu.sync_copy(idx_hbm.at[pl.ds(base, RPW)], idx_v)

    pltpu.make_async_copy(w_hbm, table_v, tsem).wait()

    def expand16(idx16, rows, dst_base):
        for lane in range(L):
            src = idx16[lane] * DP
            dst = dst_base + lane * D
            for cb in range(0, D, L * 8):
                vals = [table_v[pl.ds(src + cb + k * L, L)] for k in range(8)]
                for k in range(8):
                    rows[pl.ds(dst + cb + k * L, L)] = vals[k]

    def compute(j, rows):
        @plsc.parallel_loop(0, C, unroll=2)
        def _(r):
            t = idx_v[pl.ds(j * C + r, L)][0]
            src = t * DP
            dst = r * D
            for cb in range(0, D, L * 8):
                vals = [table_v[pl.ds(src + cb + k * L, L)] for k in range(8)]
                for k in range(8):
                    rows[pl.ds(dst + cb + k * L, L)] = vals[k]

    def scatter_start(j, rows, sem):
        pltpu.async_copy(rows, out_hbm.at[pl.ds((base + j * C) * D, C * D)], sem)

    def scatter_wait(rows, sem):
        pltpu.make_async_copy(rows, out_hbm.at[pl.ds(base * D, C * D)], sem).wait()

    def step(t, carry):
        j0 = 2 * t
        j1 = j0 + 1

        @pl.when(t > 0)
        def _():
            scatter_wait(rows0, o0)

        compute(j0, rows0)
        scatter_start(j0, rows0, o0)

        @pl.when(t > 0)
        def _():
            scatter_wait(rows1, o1)

        compute(j1, rows1)
        scatter_start(j1, rows1, o1)
        return carry

    lax.fori_loop(0, nval // 2, step, 0)
    scatter_wait(rows0, o0)
    scatter_wait(rows1, o1)

    @pl.when(wid == NW - 1)
    def _():
        tbase = (LAST_N // C) * C
        for g in range(TAIL // L):
            idx16 = idx_v[pl.ds(tbase + g * L, L)]
            expand16(idx16, rows0, g * L * D)
        pltpu.async_copy(
            rows0.at[pl.ds(0, TAIL * D)],
            out_hbm.at[pl.ds((base + tbase) * D, TAIL * D)],
            o0,
        )
        pltpu.make_async_copy(
            rows0.at[pl.ds(0, TAIL * D)],
            out_hbm.at[pl.ds((base + tbase) * D, TAIL * D)],
            o0,
        ).wait()


@jax.jit
def _emb(idx, w):
    mesh = plsc.VectorSubcoreMesh(core_axis_name="c", subcore_axis_name="s")
    f = functools.partial(
        pl.kernel,
        mesh=mesh,
        out_type=jax.ShapeDtypeStruct((B * D,), jnp.float32),
        scratch_types=[
            pltpu.VMEM((N_TYPES * DP,), jnp.float32),
            pltpu.VMEM((RPW,), jnp.int32),
            pltpu.VMEM((C * D,), jnp.float32),
            pltpu.VMEM((C * D,), jnp.float32),
            pltpu.SemaphoreType.DMA,
            pltpu.SemaphoreType.DMA,
            pltpu.SemaphoreType.DMA,
        ],
    )(_emb_body)
    return f(idx, w)


def kernel(atom_numbers, W):
    idx = jnp.squeeze(atom_numbers, axis=-1)
    w_padded = jnp.pad(W, ((0, 0), (0, L))).reshape(-1)
    out = _emb(idx, w_padded)
    return out.reshape(B, D)


# speedup vs baseline: 2.9182x; 2.8967x over previous
import functools

import jax
import jax.numpy as jnp
from jax import lax
from jax.experimental import pallas as pl
from jax.experimental.pallas import tpu as pltpu
from jax.experimental.pallas import tpu_sc as plsc

N_TYPES = 100
D = 512
B = 100000
NC = 2
NS = 16
NW = NC * NS
C = 64
NSLOT = 50
RPW = NSLOT * C
LAST_N = B - (NW - 1) * RPW
TAIL = LAST_N - (LAST_N // C) * C
L = 16


def _emb_body(idx_hbm, w_hbm, out_hbm, table_v, idx_v, rows0, rows1, tsem, o0, o1):
    wid = lax.axis_index("s") * NC + lax.axis_index("c")
    base = wid * RPW
    nval = jnp.where(wid == NW - 1, LAST_N // C, NSLOT)

    pltpu.async_copy(w_hbm, table_v, tsem)

    @pl.when(wid == NW - 1)
    def _():
        pltpu.sync_copy(idx_hbm.at[pl.ds(base, LAST_N)], idx_v.at[pl.ds(0, LAST_N)])

    @pl.when(wid != NW - 1)
    def _():
        pltpu.sync_copy(idx_hbm.at[pl.ds(base, RPW)], idx_v)

    pltpu.make_async_copy(w_hbm, table_v, tsem).wait()

    def copy_row(src, rows, r):
        for cb in range(0, D, L * 8):
            vals = [table_v[pl.ds(src + cb + k * L, L)] for k in range(8)]
            for k in range(8):
                rows[r, pl.ds(cb + k * L, L)] = vals[k]

    def compute(j, rows):
        @plsc.parallel_loop(0, C, unroll=2)
        def _(r):
            t = idx_v[pl.ds(j * C + r, L)][0]
            copy_row(t * D, rows, r)

    def scatter_start(j, rows, sem):
        pltpu.async_copy(rows, out_hbm.at[pl.ds(base + j * C, C)], sem)

    def scatter_wait(rows, sem):
        pltpu.make_async_copy(rows, out_hbm.at[pl.ds(base, C)], sem).wait()

    def step(t, carry):
        j0 = 2 * t
        j1 = j0 + 1

        @pl.when(t > 0)
        def _():
            scatter_wait(rows0, o0)

        compute(j0, rows0)
        scatter_start(j0, rows0, o0)

        @pl.when(t > 0)
        def _():
            scatter_wait(rows1, o1)

        compute(j1, rows1)
        scatter_start(j1, rows1, o1)
        return carry

    lax.fori_loop(0, nval // 2, step, 0)
    scatter_wait(rows0, o0)
    scatter_wait(rows1, o1)

    @pl.when(wid == NW - 1)
    def _():
        tbase = (LAST_N // C) * C
        for g in range(TAIL // L):
            idx16 = idx_v[pl.ds(tbase + g * L, L)]
            for lane in range(L):
                copy_row(idx16[lane] * D, rows0, g * L + lane)
        pltpu.async_copy(
            rows0.at[pl.ds(0, TAIL)],
            out_hbm.at[pl.ds(base + tbase, TAIL)],
            o0,
        )
        pltpu.make_async_copy(
            rows0.at[pl.ds(0, TAIL)],
            out_hbm.at[pl.ds(base + tbase, TAIL)],
            o0,
        ).wait()


@jax.jit
def _emb(idx, w):
    mesh = plsc.VectorSubcoreMesh(core_axis_name="c", subcore_axis_name="s")
    f = functools.partial(
        pl.kernel,
        mesh=mesh,
        out_type=jax.ShapeDtypeStruct((B, D), jnp.float32),
        scratch_types=[
            pltpu.VMEM((N_TYPES * D,), jnp.float32),
            pltpu.VMEM((RPW,), jnp.int32),
            pltpu.VMEM((C, D), jnp.float32),
            pltpu.VMEM((C, D), jnp.float32),
            pltpu.SemaphoreType.DMA,
            pltpu.SemaphoreType.DMA,
            pltpu.SemaphoreType.DMA,
        ],
    )(_emb_body)
    return f(idx, w)


def kernel(atom_numbers, W):
    idx = jnp.squeeze(atom_numbers, axis=-1)
    return _emb(idx, W.reshape(-1))


# speedup vs baseline: 3.0516x; 1.0457x over previous
import functools

import jax
import jax.numpy as jnp
from jax import lax
from jax.experimental import pallas as pl
from jax.experimental.pallas import tpu as pltpu
from jax.experimental.pallas import tpu_sc as plsc

N_TYPES = 100
D = 512
B = 100000
NC = 2
NS = 16
NW = NC * NS
C = 64
NSLOT = 50
RPW = NSLOT * C
LAST_N = B - (NW - 1) * RPW
TAIL = LAST_N - (LAST_N // C) * C
L = 16


def _emb_body(idx_hbm, w_hbm, out_hbm, table_v, idx_v, rows0, rows1, tsem, o0, o1):
    wid = lax.axis_index("s") * NC + lax.axis_index("c")
    base = wid * RPW
    nval = jnp.where(wid == NW - 1, LAST_N // C, NSLOT)

    pltpu.async_copy(w_hbm, table_v, tsem)

    @pl.when(wid == NW - 1)
    def _():
        pltpu.sync_copy(idx_hbm.at[pl.ds(base, LAST_N)], idx_v.at[pl.ds(0, LAST_N)])

    @pl.when(wid != NW - 1)
    def _():
        pltpu.sync_copy(idx_hbm.at[pl.ds(base, RPW)], idx_v)

    pltpu.make_async_copy(w_hbm, table_v, tsem).wait()

    def copy_row(src, rows, r):
        for cb in range(0, D, L * 16):
            vals = [table_v[pl.ds(src + cb + k * L, L)] for k in range(16)]
            for k in range(16):
                rows[r, pl.ds(cb + k * L, L)] = vals[k]

    def compute(j, rows):
        @plsc.parallel_loop(0, C, unroll=2)
        def _(r):
            t = idx_v[pl.ds(j * C + r, L)][0]
            copy_row(t * D, rows, r)

    def scatter_start(j, rows, sem):
        pltpu.async_copy(rows, out_hbm.at[pl.ds(base + j * C, C)], sem)

    def scatter_wait(rows, sem):
        pltpu.make_async_copy(rows, out_hbm.at[pl.ds(base, C)], sem).wait()

    def step(t, carry):
        j0 = 2 * t
        j1 = j0 + 1

        @pl.when(t > 0)
        def _():
            scatter_wait(rows0, o0)

        compute(j0, rows0)
        scatter_start(j0, rows0, o0)

        @pl.when(t > 0)
        def _():
            scatter_wait(rows1, o1)

        compute(j1, rows1)
        scatter_start(j1, rows1, o1)
        return carry

    lax.fori_loop(0, nval // 2, step, 0)
    scatter_wait(rows0, o0)
    scatter_wait(rows1, o1)

    @pl.when(wid == NW - 1)
    def _():
        tbase = (LAST_N // C) * C
        for g in range(TAIL // L):
            idx16 = idx_v[pl.ds(tbase + g * L, L)]
            for lane in range(L):
                copy_row(idx16[lane] * D, rows0, g * L + lane)
        pltpu.async_copy(
            rows0.at[pl.ds(0, TAIL)],
            out_hbm.at[pl.ds(base + tbase, TAIL)],
            o0,
        )
        pltpu.make_async_copy(
            rows0.at[pl.ds(0, TAIL)],
            out_hbm.at[pl.ds(base + tbase, TAIL)],
            o0,
        ).wait()


@jax.jit
def _emb(idx, w):
    mesh = plsc.VectorSubcoreMesh(core_axis_name="c", subcore_axis_name="s")
    f = functools.partial(
        pl.kernel,
        mesh=mesh,
        out_type=jax.ShapeDtypeStruct((B, D), jnp.float32),
        scratch_types=[
            pltpu.VMEM((N_TYPES * D,), jnp.float32),
            pltpu.VMEM((RPW,), jnp.int32),
            pltpu.VMEM((C, D), jnp.float32),
            pltpu.VMEM((C, D), jnp.float32),
            pltpu.SemaphoreType.DMA,
            pltpu.SemaphoreType.DMA,
            pltpu.SemaphoreType.DMA,
        ],
    )(_emb_body)
    return f(idx, w)


def kernel(atom_numbers, W):
    idx = jnp.squeeze(atom_numbers, axis=-1)
    return _emb(idx, W.reshape(-1))
